# baseline (speedup 1.0000x reference)
import functools

import jax
import jax.numpy as jnp
from jax import lax
from jax.experimental import pallas as pl
from jax.experimental.pallas import tpu as pltpu
from jax.experimental.pallas import tpu_sc as plsc

_TABLE_SIZE = 2 ** 19
_N_LEVELS = 16
_BASE_RES = 16.0
_MAX_RES = 128.0
_N_POINTS = 524288

_PRIME1 = -1640531535
_PRIME2 = 805459861
_MASK = _TABLE_SIZE - 1

_NW = 32
_PPW = _N_POINTS // _NW
_C = 512
_NCH = _PPW // _C

_N_DENSE = 5
_DENSE_W = [6144, 8192, 12288, 18432, 24576]
_SEG = 2048


def _hash8(a0, a1, a2):
    a0c = a0 + 1
    b1 = a1 * _PRIME1
    b1c = b1 + _PRIME1
    c2 = a2 * _PRIME2
    c2c = c2 + _PRIME2
    e00 = a0 ^ b1
    e01 = a0 ^ b1c
    e10 = a0c ^ b1
    e11 = a0c ^ b1c
    return ((e00 ^ c2) & _MASK,
            (e10 ^ c2) & _MASK,
            (e01 ^ c2) & _MASK,
            (e00 ^ c2c) & _MASK,
            (e11 ^ c2) & _MASK,
            (e10 ^ c2c) & _MASK,
            (e01 ^ c2c) & _MASK,
            (e11 ^ c2c) & _MASK)


def _weights(pd0, pd1, pd2):
    qx = 1.0 - pd0
    qy = 1.0 - pd1
    qz = 1.0 - pd2
    m00 = qx * qy
    m10 = pd0 * qy
    m01 = qx * pd1
    m11 = pd0 * pd1
    return (m00 * qz, m10 * qz, m01 * qz, m00 * pd2,
            m11 * qz, m10 * pd2, m01 * pd2, m11 * pd2)


def _sc_body(xt_hbm, table_hbm, resb_hbm, out_hbm,
             xq, pda, pdb, resv, idxa, idxb, rowsa, rowsb, outf,
             dt0, dt1, dt2, dt3, dt4, sema, semb):
    wid = lax.axis_index("s") * 2 + lax.axis_index("c")
    pltpu.sync_copy(resb_hbm, resv)
    iota = lax.broadcasted_iota(jnp.int32, (16,), 0)
    dts = (dt0, dt1, dt2, dt3, dt4)

    for li in range(_N_DENSE):
        resr = resv[pl.ds(li * 16, 16)]
        sv = resr.astype(jnp.int32) + 1
        s2 = sv * sv
        svf = resr + 1.0
        s2f = svf * svf
        dt = dts[li]

        def sbody(s, c, sv=sv, s2=s2, svf=svf, s2f=s2f, dt=dt):
            def bbody(i, cc):
                t = (s * _SEG + i * 16) + iota
                c0 = (t.astype(jnp.float32) / s2f).astype(jnp.int32)
                r = t - c0 * s2
                c0 = jnp.where(r < 0, c0 - 1, jnp.where(r >= s2, c0 + 1, c0))
                r = t - c0 * s2
                c1 = (r.astype(jnp.float32) / svf).astype(jnp.int32)
                r2 = r - c1 * sv
                c1 = jnp.where(r2 < 0, c1 - 1,
                               jnp.where(r2 >= sv, c1 + 1, c1))
                c2 = r - c1 * sv
                h = (c0 ^ (c1 * _PRIME1) ^ (c2 * _PRIME2)) & _MASK
                idxa[pl.ds(i * 16, 16)] = h
                return cc

            lax.fori_loop(0, _SEG // 16, bbody, 0)
            pltpu.async_copy(table_hbm.at[idxa.at[pl.ds(0, _SEG)]],
                             dt.at[pl.ds(s * _SEG, _SEG)], sema).wait()
            return c

        lax.fori_loop(0, _DENSE_W[li] // _SEG, sbody, 0)

    def hash_level(l, idxr, pdr, resr):
        def hbody(i, c):
            p0 = i * 16
            x0 = xq[pl.ds(p0, 16)]
            x1 = xq[pl.ds(_C + p0, 16)]
            x2 = xq[pl.ds(2 * _C + p0, 16)]
            xs0 = x0 * resr
            xs1 = x1 * resr
            xs2 = x2 * resr
            a0 = xs0.astype(jnp.int32)
            a1 = xs1.astype(jnp.int32)
            a2 = xs2.astype(jnp.int32)
            pdr[pl.ds(p0, 16)] = xs0 - a0.astype(jnp.float32)
            pdr[pl.ds(_C + p0, 16)] = xs1 - a1.astype(jnp.float32)
            pdr[pl.ds(2 * _C + p0, 16)] = xs2 - a2.astype(jnp.float32)
            hs = _hash8(a0, a1, a2)
            for jc in range(8):
                idxr[pl.ds(jc * _C + p0, 16)] = hs[jc]
            return c

        lax.fori_loop(0, _C // 16, hbody, 0)

    def fire(idxr, rowsr, sem):
        return pltpu.make_async_copy(table_hbm.at[idxr], rowsr, sem)

    def combine_level(l, rowsr, pdr):
        def cbody(i, c):
            p0 = i * 16
            pd0 = pdr[pl.ds(p0, 16)]
            pd1 = pdr[pl.ds(_C + p0, 16)]
            pd2 = pdr[pl.ds(2 * _C + p0, 16)]
            ws = _weights(pd0, pd1, pd2)
            acc0 = jnp.zeros((16,), jnp.float32)
            acc1 = jnp.zeros((16,), jnp.float32)
            for jc in range(8):
                w32 = rowsr[pl.ds(jc * _C + p0, 16)]
                wb = plsc.bitcast(w32, jnp.bfloat16)
                v0, v1 = plsc.unpack(wb, format=plsc.PackFormat.INTERLEAVED,
                                     preferred_element_type=jnp.float32)
                acc0 = acc0 + ws[jc] * v0
                acc1 = acc1 + ws[jc] * v1
            oidx = (p0 + iota) * 32 + 2 * l
            plsc.store_scatter(outf, [oidx], acc0)
            plsc.store_scatter(outf, [oidx + 1], acc1)
            return c

        lax.fori_loop(0, _C // 16, cbody, 0)

    def dense_level(li, resr):
        dt = dts[li]
        sv = resr.astype(jnp.int32) + 1
        s2 = sv * sv

        def dbody(i, c):
            p0 = i * 16
            x0 = xq[pl.ds(p0, 16)]
            x1 = xq[pl.ds(_C + p0, 16)]
            x2 = xq[pl.ds(2 * _C + p0, 16)]
            xs0 = x0 * resr
            xs1 = x1 * resr
            xs2 = x2 * resr
            a0 = xs0.astype(jnp.int32)
            a1 = xs1.astype(jnp.int32)
            a2 = xs2.astype(jnp.int32)
            pd0 = xs0 - a0.astype(jnp.float32)
            pd1 = xs1 - a1.astype(jnp.float32)
            pd2 = xs2 - a2.astype(jnp.float32)
            ws = _weights(pd0, pd1, pd2)
            t000 = (a0 * sv + a1) * sv + a2
            tj = (t000, t000 + s2, t000 + sv, t000 + 1,
                  t000 + s2 + sv, t000 + s2 + 1, t000 + sv + 1,
                  t000 + s2 + sv + 1)
            acc0 = jnp.zeros((16,), jnp.float32)
            acc1 = jnp.zeros((16,), jnp.float32)
            for jc in range(8):
                w32 = plsc.load_gather(dt, [tj[jc]])
                wb = plsc.bitcast(w32, jnp.bfloat16)
                v0, v1 = plsc.unpack(wb, format=plsc.PackFormat.INTERLEAVED,
                                     preferred_element_type=jnp.float32)
                acc0 = acc0 + ws[jc] * v0
                acc1 = acc1 + ws[jc] * v1
            oidx = (p0 + iota) * 32 + 2 * li
            plsc.store_scatter(outf, [oidx], acc0)
            plsc.store_scatter(outf, [oidx + 1], acc1)
            return c

        lax.fori_loop(0, _C // 16, dbody, 0)

    def chunk_body(ci, carry):
        base = wid * _PPW + ci * _C
        for d in range(3):
            pltpu.sync_copy(xt_hbm.at[pl.ds(d * _N_POINTS + base, _C)],
                            xq.at[pl.ds(d * _C, _C)])

        hash_level(_N_DENSE, idxa, pda, resv[pl.ds(_N_DENSE * 16, 16)])
        fire(idxa, rowsa, sema).start()

        for li in range(_N_DENSE):
            dense_level(li, resv[pl.ds(li * 16, 16)])

        def pair_body(j, c):
            l = _N_DENSE + 1 + 2 * j
            hash_level(l, idxb, pdb, resv[pl.ds(l * 16, 16)])
            fire(idxb, rowsb, semb).start()
            fire(idxa, rowsa, sema).wait()
            combine_level(l - 1, rowsa, pda)
            hash_level(l + 1, idxa, pda, resv[pl.ds((l + 1) * 16, 16)])
            fire(idxa, rowsa, sema).start()
            fire(idxb, rowsb, semb).wait()
            combine_level(l, rowsb, pdb)
            return c

        lax.fori_loop(0, (_N_LEVELS - _N_DENSE - 1) // 2, pair_body, 0)

        fire(idxa, rowsa, sema).wait()
        combine_level(_N_LEVELS - 1, rowsa, pda)

        pltpu.sync_copy(outf, out_hbm.at[pl.ds(base * 32, _C * 32)])
        return carry

    lax.fori_loop(0, _NCH, chunk_body, 0)


@functools.partial(pl.kernel,
                   out_type=jax.ShapeDtypeStruct((_N_POINTS * 32,), jnp.float32),
                   compiler_params=pltpu.CompilerParams(
                       needs_layout_passes=False),
                   mesh=plsc.VectorSubcoreMesh(core_axis_name="c",
                                               subcore_axis_name="s"),
                   scratch_types=[
                       pltpu.VMEM((3 * _C,), jnp.float32),
                       pltpu.VMEM((3 * _C,), jnp.float32),
                       pltpu.VMEM((3 * _C,), jnp.float32),
                       pltpu.VMEM((_N_LEVELS * 16,), jnp.float32),
                       pltpu.VMEM((8 * _C,), jnp.int32),
                       pltpu.VMEM((8 * _C,), jnp.int32),
                       pltpu.VMEM((8 * _C,), jnp.int32),
                       pltpu.VMEM((8 * _C,), jnp.int32),
                       pltpu.VMEM((_C * 32,), jnp.float32),
                       pltpu.VMEM((_DENSE_W[0],), jnp.int32),
                       pltpu.VMEM((_DENSE_W[1],), jnp.int32),
                       pltpu.VMEM((_DENSE_W[2],), jnp.int32),
                       pltpu.VMEM((_DENSE_W[3],), jnp.int32),
                       pltpu.VMEM((_DENSE_W[4],), jnp.int32),
                       pltpu.SemaphoreType.DMA,
                       pltpu.SemaphoreType.DMA,
                   ])
def _sc_encode(xt, table, resb, out, *rest):
    _sc_body(xt, table, resb, out, *rest)


def kernel(x, hashtable):
    xt = x.T.reshape(-1)
    growth = jnp.exp((jnp.log(jnp.float32(_MAX_RES))
                      - jnp.log(jnp.float32(_BASE_RES))) / (_N_LEVELS - 1))
    res = jnp.stack([jnp.floor(jnp.float32(_BASE_RES) * growth ** i)
                     for i in range(_N_LEVELS)])
    resb = jnp.broadcast_to(res[:, None], (_N_LEVELS, 16)).reshape(-1)
    tblq = jax.lax.bitcast_convert_type(hashtable.astype(jnp.bfloat16),
                                        jnp.int32)
    out = _sc_encode(xt, tblq, resb)
    return out.reshape(_N_POINTS, 32)


# speedup vs baseline: 1.2234x; 1.2234x over previous
import functools

import jax
import jax.numpy as jnp
from jax import lax
from jax.experimental import pallas as pl
from jax.experimental.pallas import tpu as pltpu
from jax.experimental.pallas import tpu_sc as plsc

_TABLE_SIZE = 2 ** 19
_N_LEVELS = 16
_BASE_RES = 16.0
_MAX_RES = 128.0
_N_POINTS = 524288

_PRIME1 = -1640531535
_PRIME2 = 805459861
_MASK = _TABLE_SIZE - 1

_NW = 32
_PPW = _N_POINTS // _NW
_C = 512
_NCH = _PPW // _C

_N_DENSE = 5
_DENSE_W = [6144, 8192, 12288, 18432, 24576]
_SEG = 2048
_N_SHARED = 3
_SHARED_W = [40960, 55296, 86016]


def _hash8(a0, a1, a2):
    a0c = a0 + 1
    b1 = a1 * _PRIME1
    b1c = b1 + _PRIME1
    c2 = a2 * _PRIME2
    c2c = c2 + _PRIME2
    e00 = a0 ^ b1
    e01 = a0 ^ b1c
    e10 = a0c ^ b1
    e11 = a0c ^ b1c
    return ((e00 ^ c2) & _MASK,
            (e10 ^ c2) & _MASK,
            (e01 ^ c2) & _MASK,
            (e00 ^ c2c) & _MASK,
            (e11 ^ c2) & _MASK,
            (e10 ^ c2c) & _MASK,
            (e01 ^ c2c) & _MASK,
            (e11 ^ c2c) & _MASK)


def _weights(pd0, pd1, pd2):
    qx = 1.0 - pd0
    qy = 1.0 - pd1
    qz = 1.0 - pd2
    m00 = qx * qy
    m10 = pd0 * qy
    m01 = qx * pd1
    m11 = pd0 * pd1
    return (m00 * qz, m10 * qz, m01 * qz, m00 * pd2,
            m11 * qz, m10 * pd2, m01 * pd2, m11 * pd2)


def _sc_body(xt_hbm, table_hbm, resb_hbm, out_hbm,
             xq, pda, pdb, resv, idxa, idxb, rowsa, rowsb, outf,
             dt0, dt1, dt2, dt3, dt4, sdt0, sdt1, sdt2, sema, semb):
    sid = lax.axis_index("s")
    wid = sid * 2 + lax.axis_index("c")
    pltpu.sync_copy(resb_hbm, resv)
    iota = lax.broadcasted_iota(jnp.int32, (16,), 0)
    dts = (dt0, dt1, dt2, dt3, dt4)
    sdts = (sdt0, sdt1, sdt2)

    def cell_hash(t, sv, s2, svf, s2f):
        c0 = (t.astype(jnp.float32) / s2f).astype(jnp.int32)
        r = t - c0 * s2
        c0 = jnp.where(r < 0, c0 - 1, jnp.where(r >= s2, c0 + 1, c0))
        r = t - c0 * s2
        c1 = (r.astype(jnp.float32) / svf).astype(jnp.int32)
        r2 = r - c1 * sv
        c1 = jnp.where(r2 < 0, c1 - 1, jnp.where(r2 >= sv, c1 + 1, c1))
        c2 = r - c1 * sv
        return (c0 ^ (c1 * _PRIME1) ^ (c2 * _PRIME2)) & _MASK

    for li in range(_N_DENSE):
        resr = resv[pl.ds(li * 16, 16)]
        sv = resr.astype(jnp.int32) + 1
        s2 = sv * sv
        svf = resr + 1.0
        s2f = svf * svf
        dt = dts[li]

        def sbody(s, c, sv=sv, s2=s2, svf=svf, s2f=s2f, dt=dt):
            def bbody(i, cc):
                t = (s * _SEG + i * 16) + iota
                idxa[pl.ds(i * 16, 16)] = cell_hash(t, sv, s2, svf, s2f)
                return cc

            lax.fori_loop(0, _SEG // 16, bbody, 0)
            pltpu.async_copy(table_hbm.at[idxa.at[pl.ds(0, _SEG)]],
                             dt.at[pl.ds(s * _SEG, _SEG)], sema).wait()
            return c

        lax.fori_loop(0, _DENSE_W[li] // _SEG, sbody, 0)

    for li in range(_N_SHARED):
        lvl = _N_DENSE + li
        resr = resv[pl.ds(lvl * 16, 16)]
        sv = resr.astype(jnp.int32) + 1
        s2 = sv * sv
        svf = resr + 1.0
        s2f = svf * svf
        sdt = sdts[li]
        nseg = _SHARED_W[li] // _SEG
        npt = -(-nseg // 16)
        lo = sid * npt
        hi = jnp.minimum(lo + npt, nseg)

        def sbody2(s, c, sv=sv, s2=s2, svf=svf, s2f=s2f, sdt=sdt):
            def bbody(i, cc):
                t = (s * _SEG + i * 16) + iota
                idxa[pl.ds(i * 16, 16)] = cell_hash(t, sv, s2, svf, s2f)
                return cc

            lax.fori_loop(0, _SEG // 16, bbody, 0)
            pltpu.async_copy(table_hbm.at[idxa.at[pl.ds(0, _SEG)]],
                             rowsa.at[pl.ds(0, _SEG)], sema).wait()
            pltpu.sync_copy(rowsa.at[pl.ds(0, _SEG)],
                            sdt.at[pl.ds(s * _SEG, _SEG)])
            return c

        lax.fori_loop(lo, hi, sbody2, 0)

    plsc.subcore_barrier()

    def hash_level(l, idxr, pdr, resr):
        def hbody(i, c):
            p0 = i * 16
            x0 = xq[pl.ds(p0, 16)]
            x1 = xq[pl.ds(_C + p0, 16)]
            x2 = xq[pl.ds(2 * _C + p0, 16)]
            xs0 = x0 * resr
            xs1 = x1 * resr
            xs2 = x2 * resr
            a0 = xs0.astype(jnp.int32)
            a1 = xs1.astype(jnp.int32)
            a2 = xs2.astype(jnp.int32)
            pdr[pl.ds(p0, 16)] = xs0 - a0.astype(jnp.float32)
            pdr[pl.ds(_C + p0, 16)] = xs1 - a1.astype(jnp.float32)
            pdr[pl.ds(2 * _C + p0, 16)] = xs2 - a2.astype(jnp.float32)
            hs = _hash8(a0, a1, a2)
            for jc in range(8):
                idxr[pl.ds(jc * _C + p0, 16)] = hs[jc]
            return c

        lax.fori_loop(0, _C // 16, hbody, 0)

    def fire(srcr, idxr, rowsr, sem):
        return pltpu.make_async_copy(srcr.at[idxr], rowsr, sem)

    def combine_level(l, rowsr, pdr):
        def cbody(i, c):
            p0 = i * 16
            pd0 = pdr[pl.ds(p0, 16)]
            pd1 = pdr[pl.ds(_C + p0, 16)]
            pd2 = pdr[pl.ds(2 * _C + p0, 16)]
            ws = _weights(pd0, pd1, pd2)
            acc0 = jnp.zeros((16,), jnp.float32)
            acc1 = jnp.zeros((16,), jnp.float32)
            for jc in range(8):
                w32 = rowsr[pl.ds(jc * _C + p0, 16)]
                wb = plsc.bitcast(w32, jnp.bfloat16)
                v0, v1 = plsc.unpack(wb, format=plsc.PackFormat.INTERLEAVED,
                                     preferred_element_type=jnp.float32)
                acc0 = acc0 + ws[jc] * v0
                acc1 = acc1 + ws[jc] * v1
            oidx = (p0 + iota) * 32 + 2 * l
            plsc.store_scatter(outf, [oidx], acc0)
            plsc.store_scatter(outf, [oidx + 1], acc1)
            return c

        lax.fori_loop(0, _C // 16, cbody, 0)

    def didx_level(idxr, pdr, resr):
        sv = resr.astype(jnp.int32) + 1
        s2 = sv * sv

        def hbody(i, c):
            p0 = i * 16
            x0 = xq[pl.ds(p0, 16)]
            x1 = xq[pl.ds(_C + p0, 16)]
            x2 = xq[pl.ds(2 * _C + p0, 16)]
            xs0 = x0 * resr
            xs1 = x1 * resr
            xs2 = x2 * resr
            a0 = xs0.astype(jnp.int32)
            a1 = xs1.astype(jnp.int32)
            a2 = xs2.astype(jnp.int32)
            pdr[pl.ds(p0, 16)] = xs0 - a0.astype(jnp.float32)
            pdr[pl.ds(_C + p0, 16)] = xs1 - a1.astype(jnp.float32)
            pdr[pl.ds(2 * _C + p0, 16)] = xs2 - a2.astype(jnp.float32)
            t000 = (a0 * sv + a1) * sv + a2
            tj = (t000, t000 + s2, t000 + sv, t000 + 1,
                  t000 + s2 + sv, t000 + s2 + 1, t000 + sv + 1,
                  t000 + s2 + sv + 1)
            for jc in range(8):
                idxr[pl.ds(jc * _C + p0, 16)] = tj[jc]
            return c

        lax.fori_loop(0, _C // 16, hbody, 0)

    def dense_level(li, resr):
        dt = dts[li]
        sv = resr.astype(jnp.int32) + 1
        s2 = sv * sv

        def dbody(i, c):
            p0 = i * 16
            x0 = xq[pl.ds(p0, 16)]
            x1 = xq[pl.ds(_C + p0, 16)]
            x2 = xq[pl.ds(2 * _C + p0, 16)]
            xs0 = x0 * resr
            xs1 = x1 * resr
            xs2 = x2 * resr
            a0 = xs0.astype(jnp.int32)
            a1 = xs1.astype(jnp.int32)
            a2 = xs2.astype(jnp.int32)
            pd0 = xs0 - a0.astype(jnp.float32)
            pd1 = xs1 - a1.astype(jnp.float32)
            pd2 = xs2 - a2.astype(jnp.float32)
            ws = _weights(pd0, pd1, pd2)
            t000 = (a0 * sv + a1) * sv + a2
            tj = (t000, t000 + s2, t000 + sv, t000 + 1,
                  t000 + s2 + sv, t000 + s2 + 1, t000 + sv + 1,
                  t000 + s2 + sv + 1)
            acc0 = jnp.zeros((16,), jnp.float32)
            acc1 = jnp.zeros((16,), jnp.float32)
            for jc in range(8):
                w32 = plsc.load_gather(dt, [tj[jc]])
                wb = plsc.bitcast(w32, jnp.bfloat16)
                v0, v1 = plsc.unpack(wb, format=plsc.PackFormat.INTERLEAVED,
                                     preferred_element_type=jnp.float32)
                acc0 = acc0 + ws[jc] * v0
                acc1 = acc1 + ws[jc] * v1
            oidx = (p0 + iota) * 32 + 2 * li
            plsc.store_scatter(outf, [oidx], acc0)
            plsc.store_scatter(outf, [oidx + 1], acc1)
            return c

        lax.fori_loop(0, _C // 16, dbody, 0)

    def chunk_body(ci, carry):
        base = wid * _PPW + ci * _C
        for d in range(3):
            pltpu.sync_copy(xt_hbm.at[pl.ds(d * _N_POINTS + base, _C)],
                            xq.at[pl.ds(d * _C, _C)])

        didx_level(idxa, pda, resv[pl.ds(5 * 16, 16)])
        fire(sdt0, idxa, rowsa, sema).start()
        dense_level(0, resv[pl.ds(0, 16)])
        dense_level(1, resv[pl.ds(16, 16)])
        didx_level(idxb, pdb, resv[pl.ds(6 * 16, 16)])
        fire(sdt1, idxb, rowsb, semb).start()
        fire(sdt0, idxa, rowsa, sema).wait()
        combine_level(5, rowsa, pda)
        dense_level(2, resv[pl.ds(2 * 16, 16)])
        didx_level(idxa, pda, resv[pl.ds(7 * 16, 16)])
        fire(sdt2, idxa, rowsa, sema).start()
        fire(sdt1, idxb, rowsb, semb).wait()
        combine_level(6, rowsb, pdb)
        dense_level(3, resv[pl.ds(3 * 16, 16)])
        hash_level(8, idxb, pdb, resv[pl.ds(8 * 16, 16)])
        fire(table_hbm, idxb, rowsb, semb).start()
        fire(sdt2, idxa, rowsa, sema).wait()
        combine_level(7, rowsa, pda)
        dense_level(4, resv[pl.ds(4 * 16, 16)])

        def pair_body(j, c):
            l = 9 + 2 * j
            hash_level(l, idxa, pda, resv[pl.ds(l * 16, 16)])
            fire(table_hbm, idxa, rowsa, sema).start()
            fire(table_hbm, idxb, rowsb, semb).wait()
            combine_level(l - 1, rowsb, pdb)
            hash_level(l + 1, idxb, pdb, resv[pl.ds((l + 1) * 16, 16)])
            fire(table_hbm, idxb, rowsb, semb).start()
            fire(table_hbm, idxa, rowsa, sema).wait()
            combine_level(l, rowsa, pda)
            return c

        lax.fori_loop(0, 3, pair_body, 0)

        hash_level(15, idxa, pda, resv[pl.ds(15 * 16, 16)])
        fire(table_hbm, idxa, rowsa, sema).start()
        fire(table_hbm, idxb, rowsb, semb).wait()
        combine_level(14, rowsb, pdb)
        fire(table_hbm, idxa, rowsa, sema).wait()
        combine_level(15, rowsa, pda)

        pltpu.sync_copy(outf, out_hbm.at[pl.ds(base * 32, _C * 32)])
        return carry

    lax.fori_loop(0, _NCH, chunk_body, 0)


@functools.partial(pl.kernel,
                   out_type=jax.ShapeDtypeStruct((_N_POINTS * 32,), jnp.float32),
                   compiler_params=pltpu.CompilerParams(
                       needs_layout_passes=False),
                   mesh=plsc.VectorSubcoreMesh(core_axis_name="c",
                                               subcore_axis_name="s"),
                   scratch_types=[
                       pltpu.VMEM((3 * _C,), jnp.float32),
                       pltpu.VMEM((3 * _C,), jnp.float32),
                       pltpu.VMEM((3 * _C,), jnp.float32),
                       pltpu.VMEM((_N_LEVELS * 16,), jnp.float32),
                       pltpu.VMEM((8 * _C,), jnp.int32),
                       pltpu.VMEM((8 * _C,), jnp.int32),
                       pltpu.VMEM((8 * _C,), jnp.int32),
                       pltpu.VMEM((8 * _C,), jnp.int32),
                       pltpu.VMEM((_C * 32,), jnp.float32),
                       pltpu.VMEM((_DENSE_W[0],), jnp.int32),
                       pltpu.VMEM((_DENSE_W[1],), jnp.int32),
                       pltpu.VMEM((_DENSE_W[2],), jnp.int32),
                       pltpu.VMEM((_DENSE_W[3],), jnp.int32),
                       pltpu.VMEM((_DENSE_W[4],), jnp.int32),
                       pltpu.VMEM_SHARED((_SHARED_W[0],), jnp.int32),
                       pltpu.VMEM_SHARED((_SHARED_W[1],), jnp.int32),
                       pltpu.VMEM_SHARED((_SHARED_W[2],), jnp.int32),
                       pltpu.SemaphoreType.DMA,
                       pltpu.SemaphoreType.DMA,
                   ])
def _sc_encode(xt, table, resb, out, *rest):
    _sc_body(xt, table, resb, out, *rest)


def kernel(x, hashtable):
    xt = x.T.reshape(-1)
    growth = jnp.exp((jnp.log(jnp.float32(_MAX_RES))
                      - jnp.log(jnp.float32(_BASE_RES))) / (_N_LEVELS - 1))
    res = jnp.stack([jnp.floor(jnp.float32(_BASE_RES) * growth ** i)
                     for i in range(_N_LEVELS)])
    resb = jnp.broadcast_to(res[:, None], (_N_LEVELS, 16)).reshape(-1)
    tblq = jax.lax.bitcast_convert_type(hashtable.astype(jnp.bfloat16),
                                        jnp.int32)
    out = _sc_encode(xt, tblq, resb)
    return out.reshape(_N_POINTS, 32)


# speedup vs baseline: 1.2243x; 1.0007x over previous
import functools

import jax
import jax.numpy as jnp
from jax import lax
from jax.experimental import pallas as pl
from jax.experimental.pallas import tpu as pltpu
from jax.experimental.pallas import tpu_sc as plsc

_TABLE_SIZE = 2 ** 19
_N_LEVELS = 16
_BASE_RES = 16.0
_MAX_RES = 128.0
_N_POINTS = 524288

_PRIME1 = -1640531535
_PRIME2 = 805459861
_MASK = _TABLE_SIZE - 1

_NW = 32
_PPW = _N_POINTS // _NW
_C = 512
_NCH = _PPW // _C

_N_DENSE = 5
_DENSE_W = [6144, 8192, 12288, 18432, 24576]
_SEG = 2048
_N_SHARED = 4
_SHARED_W = [40960, 55296, 86016, 126976]


def _hash8(a0, a1, a2):
    a0c = a0 + 1
    b1 = a1 * _PRIME1
    b1c = b1 + _PRIME1
    c2 = a2 * _PRIME2
    c2c = c2 + _PRIME2
    e00 = a0 ^ b1
    e01 = a0 ^ b1c
    e10 = a0c ^ b1
    e11 = a0c ^ b1c
    return ((e00 ^ c2) & _MASK,
            (e10 ^ c2) & _MASK,
            (e01 ^ c2) & _MASK,
            (e00 ^ c2c) & _MASK,
            (e11 ^ c2) & _MASK,
            (e10 ^ c2c) & _MASK,
            (e01 ^ c2c) & _MASK,
            (e11 ^ c2c) & _MASK)


def _weights(pd0, pd1, pd2):
    qx = 1.0 - pd0
    qy = 1.0 - pd1
    qz = 1.0 - pd2
    m00 = qx * qy
    m10 = pd0 * qy
    m01 = qx * pd1
    m11 = pd0 * pd1
    return (m00 * qz, m10 * qz, m01 * qz, m00 * pd2,
            m11 * qz, m10 * pd2, m01 * pd2, m11 * pd2)


def _sc_body(xt_hbm, table_hbm, resb_hbm, out_hbm,
             xq, pda, pdb, resv, idxa, idxb, rowsa, rowsb, outf,
             dt0, dt1, dt2, dt3, dt4, sdt0, sdt1, sdt2, sdt3, sema, semb):
    sid = lax.axis_index("s")
    wid = sid * 2 + lax.axis_index("c")
    pltpu.sync_copy(resb_hbm, resv)
    iota = lax.broadcasted_iota(jnp.int32, (16,), 0)
    dts = (dt0, dt1, dt2, dt3, dt4)
    sdts = (sdt0, sdt1, sdt2, sdt3)

    def cell_hash(t, sv, s2, svf, s2f):
        c0 = (t.astype(jnp.float32) / s2f).astype(jnp.int32)
        r = t - c0 * s2
        c0 = jnp.where(r < 0, c0 - 1, jnp.where(r >= s2, c0 + 1, c0))
        r = t - c0 * s2
        c1 = (r.astype(jnp.float32) / svf).astype(jnp.int32)
        r2 = r - c1 * sv
        c1 = jnp.where(r2 < 0, c1 - 1, jnp.where(r2 >= sv, c1 + 1, c1))
        c2 = r - c1 * sv
        return (c0 ^ (c1 * _PRIME1) ^ (c2 * _PRIME2)) & _MASK

    for li in range(_N_DENSE):
        resr = resv[pl.ds(li * 16, 16)]
        sv = resr.astype(jnp.int32) + 1
        s2 = sv * sv
        svf = resr + 1.0
        s2f = svf * svf
        dt = dts[li]

        def sbody(s, c, sv=sv, s2=s2, svf=svf, s2f=s2f, dt=dt):
            def bbody(i, cc):
                t = (s * _SEG + i * 16) + iota
                idxa[pl.ds(i * 16, 16)] = cell_hash(t, sv, s2, svf, s2f)
                return cc

            lax.fori_loop(0, _SEG // 16, bbody, 0)
            pltpu.async_copy(table_hbm.at[idxa.at[pl.ds(0, _SEG)]],
                             dt.at[pl.ds(s * _SEG, _SEG)], sema).wait()
            return c

        lax.fori_loop(0, _DENSE_W[li] // _SEG, sbody, 0)

    for li in range(_N_SHARED):
        lvl = _N_DENSE + li
        resr = resv[pl.ds(lvl * 16, 16)]
        sv = resr.astype(jnp.int32) + 1
        s2 = sv * sv
        svf = resr + 1.0
        s2f = svf * svf
        sdt = sdts[li]
        nseg = _SHARED_W[li] // _SEG
        npt = -(-nseg // 16)
        lo = sid * npt
        hi = jnp.minimum(lo + npt, nseg)

        def sbody2(s, c, sv=sv, s2=s2, svf=svf, s2f=s2f, sdt=sdt):
            def bbody(i, cc):
                t = (s * _SEG + i * 16) + iota
                idxa[pl.ds(i * 16, 16)] = cell_hash(t, sv, s2, svf, s2f)
                return cc

            lax.fori_loop(0, _SEG // 16, bbody, 0)
            pltpu.async_copy(table_hbm.at[idxa.at[pl.ds(0, _SEG)]],
                             rowsa.at[pl.ds(0, _SEG)], sema).wait()
            pltpu.sync_copy(rowsa.at[pl.ds(0, _SEG)],
                            sdt.at[pl.ds(s * _SEG, _SEG)])
            return c

        lax.fori_loop(lo, hi, sbody2, 0)

    plsc.subcore_barrier()

    def hash_level(l, idxr, pdr, resr):
        def hbody(i, c):
            p0 = i * 16
            x0 = xq[pl.ds(p0, 16)]
            x1 = xq[pl.ds(_C + p0, 16)]
            x2 = xq[pl.ds(2 * _C + p0, 16)]
            xs0 = x0 * resr
            xs1 = x1 * resr
            xs2 = x2 * resr
            a0 = xs0.astype(jnp.int32)
            a1 = xs1.astype(jnp.int32)
            a2 = xs2.astype(jnp.int32)
            pdr[pl.ds(p0, 16)] = xs0 - a0.astype(jnp.float32)
            pdr[pl.ds(_C + p0, 16)] = xs1 - a1.astype(jnp.float32)
            pdr[pl.ds(2 * _C + p0, 16)] = xs2 - a2.astype(jnp.float32)
            hs = _hash8(a0, a1, a2)
            for jc in range(8):
                idxr[pl.ds(jc * _C + p0, 16)] = hs[jc]
            return c

        lax.fori_loop(0, _C // 16, hbody, 0)

    def fire(srcr, idxr, rowsr, sem):
        return pltpu.make_async_copy(srcr.at[idxr], rowsr, sem)

    def combine_level(l, rowsr, pdr):
        def cbody(i, c):
            p0 = i * 16
            pd0 = pdr[pl.ds(p0, 16)]
            pd1 = pdr[pl.ds(_C + p0, 16)]
            pd2 = pdr[pl.ds(2 * _C + p0, 16)]
            ws = _weights(pd0, pd1, pd2)
            acc0 = jnp.zeros((16,), jnp.float32)
            acc1 = jnp.zeros((16,), jnp.float32)
            for jc in range(8):
                w32 = rowsr[pl.ds(jc * _C + p0, 16)]
                wb = plsc.bitcast(w32, jnp.bfloat16)
                v0, v1 = plsc.unpack(wb, format=plsc.PackFormat.INTERLEAVED,
                                     preferred_element_type=jnp.float32)
                acc0 = acc0 + ws[jc] * v0
                acc1 = acc1 + ws[jc] * v1
            oidx = (p0 + iota) * 32 + 2 * l
            plsc.store_scatter(outf, [oidx], acc0)
            plsc.store_scatter(outf, [oidx + 1], acc1)
            return c

        lax.fori_loop(0, _C // 16, cbody, 0)

    def didx_level(idxr, pdr, resr):
        sv = resr.astype(jnp.int32) + 1
        s2 = sv * sv

        def hbody(i, c):
            p0 = i * 16
            x0 = xq[pl.ds(p0, 16)]
            x1 = xq[pl.ds(_C + p0, 16)]
            x2 = xq[pl.ds(2 * _C + p0, 16)]
            xs0 = x0 * resr
            xs1 = x1 * resr
            xs2 = x2 * resr
            a0 = xs0.astype(jnp.int32)
            a1 = xs1.astype(jnp.int32)
            a2 = xs2.astype(jnp.int32)
            pdr[pl.ds(p0, 16)] = xs0 - a0.astype(jnp.float32)
            pdr[pl.ds(_C + p0, 16)] = xs1 - a1.astype(jnp.float32)
            pdr[pl.ds(2 * _C + p0, 16)] = xs2 - a2.astype(jnp.float32)
            t000 = (a0 * sv + a1) * sv + a2
            tj = (t000, t000 + s2, t000 + sv, t000 + 1,
                  t000 + s2 + sv, t000 + s2 + 1, t000 + sv + 1,
                  t000 + s2 + sv + 1)
            for jc in range(8):
                idxr[pl.ds(jc * _C + p0, 16)] = tj[jc]
            return c

        lax.fori_loop(0, _C // 16, hbody, 0)

    def dense_level(li, resr):
        dt = dts[li]
        sv = resr.astype(jnp.int32) + 1
        s2 = sv * sv

        def dbody(i, c):
            p0 = i * 16
            x0 = xq[pl.ds(p0, 16)]
            x1 = xq[pl.ds(_C + p0, 16)]
            x2 = xq[pl.ds(2 * _C + p0, 16)]
            xs0 = x0 * resr
            xs1 = x1 * resr
            xs2 = x2 * resr
            a0 = xs0.astype(jnp.int32)
            a1 = xs1.astype(jnp.int32)
            a2 = xs2.astype(jnp.int32)
            pd0 = xs0 - a0.astype(jnp.float32)
            pd1 = xs1 - a1.astype(jnp.float32)
            pd2 = xs2 - a2.astype(jnp.float32)
            ws = _weights(pd0, pd1, pd2)
            t000 = (a0 * sv + a1) * sv + a2
            tj = (t000, t000 + s2, t000 + sv, t000 + 1,
                  t000 + s2 + sv, t000 + s2 + 1, t000 + sv + 1,
                  t000 + s2 + sv + 1)
            acc0 = jnp.zeros((16,), jnp.float32)
            acc1 = jnp.zeros((16,), jnp.float32)
            for jc in range(8):
                w32 = plsc.load_gather(dt, [tj[jc]])
                wb = plsc.bitcast(w32, jnp.bfloat16)
                v0, v1 = plsc.unpack(wb, format=plsc.PackFormat.INTERLEAVED,
                                     preferred_element_type=jnp.float32)
                acc0 = acc0 + ws[jc] * v0
                acc1 = acc1 + ws[jc] * v1
            oidx = (p0 + iota) * 32 + 2 * li
            plsc.store_scatter(outf, [oidx], acc0)
            plsc.store_scatter(outf, [oidx + 1], acc1)
            return c

        lax.fori_loop(0, _C // 16, dbody, 0)

    def chunk_body(ci, carry):
        base = wid * _PPW + ci * _C
        for d in range(3):
            pltpu.sync_copy(xt_hbm.at[pl.ds(d * _N_POINTS + base, _C)],
                            xq.at[pl.ds(d * _C, _C)])

        didx_level(idxa, pda, resv[pl.ds(5 * 16, 16)])
        fire(sdt0, idxa, rowsa, sema).start()
        dense_level(0, resv[pl.ds(0, 16)])
        dense_level(1, resv[pl.ds(16, 16)])
        didx_level(idxb, pdb, resv[pl.ds(6 * 16, 16)])
        fire(sdt1, idxb, rowsb, semb).start()
        fire(sdt0, idxa, rowsa, sema).wait()
        combine_level(5, rowsa, pda)
        dense_level(2, resv[pl.ds(2 * 16, 16)])
        didx_level(idxa, pda, resv[pl.ds(7 * 16, 16)])
        fire(sdt2, idxa, rowsa, sema).start()
        fire(sdt1, idxb, rowsb, semb).wait()
        combine_level(6, rowsb, pdb)
        dense_level(3, resv[pl.ds(3 * 16, 16)])
        didx_level(idxb, pdb, resv[pl.ds(8 * 16, 16)])
        fire(sdt3, idxb, rowsb, semb).start()
        fire(sdt2, idxa, rowsa, sema).wait()
        combine_level(7, rowsa, pda)
        dense_level(4, resv[pl.ds(4 * 16, 16)])
        hash_level(9, idxa, pda, resv[pl.ds(9 * 16, 16)])
        fire(table_hbm, idxa, rowsa, sema).start()
        fire(sdt3, idxb, rowsb, semb).wait()
        combine_level(8, rowsb, pdb)

        def pair_body(j, c):
            l = 10 + 2 * j
            hash_level(l, idxb, pdb, resv[pl.ds(l * 16, 16)])
            fire(table_hbm, idxb, rowsb, semb).start()
            fire(table_hbm, idxa, rowsa, sema).wait()
            combine_level(l - 1, rowsa, pda)
            hash_level(l + 1, idxa, pda, resv[pl.ds((l + 1) * 16, 16)])
            fire(table_hbm, idxa, rowsa, sema).start()
            fire(table_hbm, idxb, rowsb, semb).wait()
            combine_level(l, rowsb, pdb)
            return c

        lax.fori_loop(0, 3, pair_body, 0)

        fire(table_hbm, idxa, rowsa, sema).wait()
        combine_level(15, rowsa, pda)

        pltpu.sync_copy(outf, out_hbm.at[pl.ds(base * 32, _C * 32)])
        return carry

    lax.fori_loop(0, _NCH, chunk_body, 0)


@functools.partial(pl.kernel,
                   out_type=jax.ShapeDtypeStruct((_N_POINTS * 32,), jnp.float32),
                   compiler_params=pltpu.CompilerParams(
                       needs_layout_passes=False),
                   mesh=plsc.VectorSubcoreMesh(core_axis_name="c",
                                               subcore_axis_name="s"),
                   scratch_types=[
                       pltpu.VMEM((3 * _C,), jnp.float32),
                       pltpu.VMEM((3 * _C,), jnp.float32),
                       pltpu.VMEM((3 * _C,), jnp.float32),
                       pltpu.VMEM((_N_LEVELS * 16,), jnp.float32),
                       pltpu.VMEM((8 * _C,), jnp.int32),
                       pltpu.VMEM((8 * _C,), jnp.int32),
                       pltpu.VMEM((8 * _C,), jnp.int32),
                       pltpu.VMEM((8 * _C,), jnp.int32),
                       pltpu.VMEM((_C * 32,), jnp.float32),
                       pltpu.VMEM((_DENSE_W[0],), jnp.int32),
                       pltpu.VMEM((_DENSE_W[1],), jnp.int32),
                       pltpu.VMEM((_DENSE_W[2],), jnp.int32),
                       pltpu.VMEM((_DENSE_W[3],), jnp.int32),
                       pltpu.VMEM((_DENSE_W[4],), jnp.int32),
                       pltpu.VMEM_SHARED((_SHARED_W[0],), jnp.int32),
                       pltpu.VMEM_SHARED((_SHARED_W[1],), jnp.int32),
                       pltpu.VMEM_SHARED((_SHARED_W[2],), jnp.int32),
                       pltpu.VMEM_SHARED((_SHARED_W[3],), jnp.int32),
                       pltpu.SemaphoreType.DMA,
                       pltpu.SemaphoreType.DMA,
                   ])
def _sc_encode(xt, table, resb, out, *rest):
    _sc_body(xt, table, resb, out, *rest)


def kernel(x, hashtable):
    xt = x.T.reshape(-1)
    growth = jnp.exp((jnp.log(jnp.float32(_MAX_RES))
                      - jnp.log(jnp.float32(_BASE_RES))) / (_N_LEVELS - 1))
    res = jnp.stack([jnp.floor(jnp.float32(_BASE_RES) * growth ** i)
                     for i in range(_N_LEVELS)])
    resb = jnp.broadcast_to(res[:, None], (_N_LEVELS, 16)).reshape(-1)
    tblq = jax.lax.bitcast_convert_type(hashtable.astype(jnp.bfloat16),
                                        jnp.int32)
    out = _sc_encode(xt, tblq, resb)
    return out.reshape(_N_POINTS, 32)
